# Initial kernel scaffold; baseline (speedup 1.0000x reference)
import jax
import jax.numpy as jnp
from jax.experimental import pallas as pl


def kernel(x, l_t_prev, h_t_prev, snaps_prev, noise, params):
    raise NotImplementedError("write your pallas kernel here")


# speedup vs baseline: 182.7157x; 182.7157x over previous
import jax
import jax.numpy as jnp
from jax.experimental import pallas as pl
from jax.experimental.pallas import tpu as pltpu

G = 16
K = 3
S = 2
C = 3
IMG = 224
H_G = 128
H_L = 128
STD = 0.17
HIDDEN = 256
NCLS = 1000
PAD = G * (S ** (K - 1))
XP = IMG + 2 * PAD


def _select_pool(d0, f, transposed):
    shape = (IMG, G) if transposed else (G, IMG)
    g = jax.lax.broadcasted_iota(jnp.int32, shape, 1 if transposed else 0)
    u = jax.lax.broadcasted_iota(jnp.int32, shape, 0 if transposed else 1)
    q = u - d0 - g * f
    sel = jnp.logical_and(q >= 0, q < f)
    return jnp.where(sel, jnp.float32(1.0 / f), jnp.float32(0.0))


def _chain_gcn_matrix():
    n = 7 + 1
    r = jax.lax.broadcasted_iota(jnp.int32, (n, n), 0)
    c = jax.lax.broadcasted_iota(jnp.int32, (n, n), 1)
    inv_sqrt2 = 1.0 / jnp.sqrt(jnp.float32(2.0))
    diag = jnp.where(r == c, jnp.where(r == 0, 1.0, 0.5), 0.0)
    sub = jnp.where(r == c + 1, jnp.where(r == 1, inv_sqrt2, 0.5), 0.0)
    return (diag + sub).astype(jnp.float32)


def _fwd_kernel(l_ref, x_ref, snaps_ref, noise_ref, w1p_ref, b1_ref, w2_ref,
                b2_ref, w3_ref, b3_ref, w4_ref, b4_ref, wg1_ref, bg1_ref,
                wg2_ref, bg2_ref, wl1_ref, bl1_ref, wl2_ref, bl2_ref, wb_ref,
                bb_ref, wc_ref, bc_ref,
                out_h, out_l, out_b, out_p, out_pi):
    f32 = jnp.float32

    ly = l_ref[0, 0]
    lx = l_ref[0, 1]

    def start(coord, size):
        center = 0.5 * ((coord + 1.0) * IMG)
        n = center.astype(jnp.int32)
        frac = center - n.astype(f32)
        odd = jnp.bitwise_and(n, 1)
        rnd = n + jnp.where(frac > 0.5, 1, jnp.where(frac == 0.5, odd, 0))
        st = rnd - size // 2 + size
        return jnp.clip(st, 0, IMG + size) - size

    r_blk = jax.lax.broadcasted_iota(jnp.int32, (G, G * H_G), 0)
    c_blk = jax.lax.broadcasted_iota(jnp.int32, (G, G * H_G), 1)
    diag_mask = (c_blk // H_G) == r_blk
    j_id = jax.lax.broadcasted_iota(jnp.int32, (G * H_G, H_G), 0)
    o_id = jax.lax.broadcasted_iota(jnp.int32, (G * H_G, H_G), 1)
    block_id = jnp.where(j_id % H_G == o_id, 1.0, 0.0).astype(f32)

    g1v = b1_ref[...]
    for i in range(K):
        size = G * (S ** i)
        f = size // G
        d0 = start(ly, size)
        d1 = start(lx, size)
        pr = _select_pool(d0, f, transposed=False)
        pct = _select_pool(d1, f, transposed=True)
        for c in range(C):
            pooled = jax.lax.dot(jax.lax.dot(pr, x_ref[c]), pct)
            q = jax.lax.dot(pooled, w1p_ref[i, c])
            s = jnp.sum(jnp.where(diag_mask, q, 0.0), axis=0, keepdims=True)
            g1v = g1v + jax.lax.dot(s, block_id)
    g1v = jnp.maximum(g1v, 0.0)

    l1 = jnp.maximum(w2_ref[0:1, :] * ly + w2_ref[1:2, :] * lx + b2_ref[...],
                     0.0)

    g_t = jnp.maximum(
        (jax.lax.dot(g1v, w3_ref[...]) + b3_ref[...])
        + (jax.lax.dot(l1, w4_ref[...]) + b4_ref[...]), 0.0)

    nf = jnp.concatenate([snaps_ref[...], g_t], axis=0)
    A = _chain_gcn_matrix()
    h1 = jnp.maximum(
        jax.lax.dot(A, jax.lax.dot(nf, wg1_ref[...])) + bg1_ref[...], 0.0)
    out2 = jax.lax.dot(A, jax.lax.dot(h1, wg2_ref[...])) + bg2_ref[...]
    h_t = jnp.mean(out2, axis=0, keepdims=True)
    out_h[...] = h_t

    feat = jnp.maximum(jax.lax.dot(h_t, wl1_ref[...]) + bl1_ref[...], 0.0)
    mu = jnp.tanh(jax.lax.dot(feat, wl2_ref[...]) + bl2_ref[...])
    l_pre = mu + STD * noise_ref[...]
    out_l[...] = jnp.clip(l_pre, -1.0, 1.0)
    z = (l_pre - mu) / STD
    terms = -0.5 * z * z - jnp.log(f32(STD)) - 0.5 * jnp.log(2.0 * f32(jnp.pi))
    out_pi[...] = jnp.sum(terms, axis=1, keepdims=True)

    out_b[...] = jax.lax.dot(h_t, wb_ref[...]) + bb_ref[...]

    logits = jax.lax.dot(h_t, wc_ref[...]) + bc_ref[...]
    m = jnp.max(logits, axis=1, keepdims=True)
    sh = logits - m
    out_p[...] = sh - jnp.log(jnp.sum(jnp.exp(sh), axis=1, keepdims=True))


def kernel(x, l_t_prev, h_t_prev, snaps_prev, noise, params):
    del h_t_prev
    p = params
    f32 = jnp.float32

    x0 = jnp.transpose(x[0], (2, 0, 1)).astype(f32)
    l0 = l_t_prev[0:1].astype(f32)

    w1p = (p['W1'].reshape(K, G, G, C, H_G)
           .transpose(0, 3, 2, 1, 4)
           .reshape(K, C, G, G * H_G))

    def row(v):
        return v.reshape(1, -1).astype(f32)

    out_shapes = (
        jax.ShapeDtypeStruct((1, HIDDEN), f32),
        jax.ShapeDtypeStruct((1, 2), f32),
        jax.ShapeDtypeStruct((1, 1), f32),
        jax.ShapeDtypeStruct((1, NCLS), f32),
        jax.ShapeDtypeStruct((1, 1), f32),
    )
    in_specs = [pl.BlockSpec(memory_space=pltpu.SMEM)] + [
        pl.BlockSpec(memory_space=pltpu.VMEM) for _ in range(23)]

    h_t, l_t, b_t, log_probas, log_pi = pl.pallas_call(
        _fwd_kernel,
        out_shape=out_shapes,
        in_specs=in_specs,
        out_specs=tuple(pl.BlockSpec(memory_space=pltpu.VMEM)
                        for _ in range(5)),
    )(l0, x0, snaps_prev.astype(f32), noise.astype(f32), w1p,
      row(p['b1']), p['W2'], row(p['b2']), p['W3'], row(p['b3']),
      p['W4'], row(p['b4']), p['Wg1'], row(p['bg1']), p['Wg2'], row(p['bg2']),
      p['Wl1'], row(p['bl1']), p['Wl2'], row(p['bl2']), p['Wb'], row(p['bb']),
      p['Wc'], row(p['bc']))

    return (h_t, l_t, b_t.reshape(()), log_probas, log_pi.reshape((1,)))
